# baseline (speedup 1.0000x reference)
import jax
import jax.numpy as jnp
from jax.experimental import pallas as pl
from jax.experimental.pallas import tpu as pltpu

N = 10000
F_IN = 128
F_OUT = 128
BM = 80


def _body(adj_ref, x_ref, w_ref, b_ref, out_ref, s_ref):
    @pl.when(pl.program_id(0) == 0)
    def _():
        s_ref[...] = jax.lax.dot_general(
            x_ref[...], w_ref[...],
            dimension_numbers=(((1,), (1,)), ((), ())),
            preferred_element_type=jnp.float32,
        ) + b_ref[...]

    out_ref[...] = jnp.dot(adj_ref[...], s_ref[...],
                           preferred_element_type=jnp.float32)


@jax.jit
def kernel(input, adj, W, b):
    b2 = b.reshape(1, F_OUT)
    num_m = N // BM
    output = pl.pallas_call(
        _body,
        grid=(num_m,),
        in_specs=[
            pl.BlockSpec((BM, N), lambda i: (i, 0)),
            pl.BlockSpec((N, F_IN), lambda i: (0, 0)),
            pl.BlockSpec((F_OUT, F_IN), lambda i: (0, 0)),
            pl.BlockSpec((1, F_OUT), lambda i: (0, 0)),
        ],
        out_specs=pl.BlockSpec((BM, F_OUT), lambda i: (i, 0)),
        out_shape=jax.ShapeDtypeStruct((N, F_OUT), jnp.float32),
        scratch_shapes=[pltpu.VMEM((N, F_OUT), jnp.float32)],
        compiler_params=pltpu.CompilerParams(
            dimension_semantics=("arbitrary",),
        ),
    )(adj, input, W, b2)
    return output


# speedup vs baseline: 1.3665x; 1.3665x over previous
import jax
import jax.numpy as jnp
from jax.experimental import pallas as pl
from jax.experimental.pallas import tpu as pltpu

N = 10000
F_IN = 128
F_OUT = 128
BM = 200
NSPLIT = 2


def _body(adj0_ref, adj1_ref, x_ref, w_ref, b_ref, out_ref, s_ref):
    @pl.when(pl.program_id(0) == 0)
    def _():
        s_ref[...] = jax.lax.dot_general(
            x_ref[...], w_ref[...],
            dimension_numbers=(((1,), (1,)), ((), ())),
            preferred_element_type=jnp.float32,
        ) + b_ref[...]

    out_ref[:BM, :] = jnp.dot(adj0_ref[...], s_ref[...],
                              preferred_element_type=jnp.float32)
    out_ref[BM:, :] = jnp.dot(adj1_ref[...], s_ref[...],
                              preferred_element_type=jnp.float32)


@jax.jit
def kernel(input, adj, W, b):
    b2 = b.reshape(1, F_OUT)
    num_m = N // (BM * NSPLIT)
    output = pl.pallas_call(
        _body,
        grid=(num_m,),
        in_specs=[
            pl.BlockSpec((BM, N), lambda i: (2 * i, 0)),
            pl.BlockSpec((BM, N), lambda i: (2 * i + 1, 0)),
            pl.BlockSpec((N, F_IN), lambda i: (0, 0)),
            pl.BlockSpec((F_OUT, F_IN), lambda i: (0, 0)),
            pl.BlockSpec((1, F_OUT), lambda i: (0, 0)),
        ],
        out_specs=pl.BlockSpec((BM * NSPLIT, F_OUT), lambda i: (i, 0)),
        out_shape=jax.ShapeDtypeStruct((N, F_OUT), jnp.float32),
        scratch_shapes=[pltpu.VMEM((N, F_OUT), jnp.float32)],
        compiler_params=pltpu.CompilerParams(
            dimension_semantics=("arbitrary",),
        ),
    )(adj, adj, input, W, b2)
    return output


# speedup vs baseline: 1.3678x; 1.0009x over previous
import jax
import jax.numpy as jnp
from jax.experimental import pallas as pl
from jax.experimental.pallas import tpu as pltpu

N = 10000
F_IN = 128
F_OUT = 128
BM = 400


def _body(adj_ref, x_ref, w_ref, b_ref, out_ref, s_ref):
    @pl.when(pl.program_id(0) == 0)
    def _():
        s_ref[...] = jax.lax.dot_general(
            x_ref[...], w_ref[...],
            dimension_numbers=(((1,), (1,)), ((), ())),
            preferred_element_type=jnp.float32,
        ) + b_ref[...]

    out_ref[...] = jnp.dot(adj_ref[...], s_ref[...],
                           preferred_element_type=jnp.float32)


@jax.jit
def kernel(input, adj, W, b):
    b2 = b.reshape(1, F_OUT)
    num_m = N // BM
    output = pl.pallas_call(
        _body,
        grid=(num_m,),
        in_specs=[
            pl.BlockSpec((BM, N), lambda i: (i, 0)),
            pl.BlockSpec((N, F_IN), lambda i: (0, 0)),
            pl.BlockSpec((F_OUT, F_IN), lambda i: (0, 0)),
            pl.BlockSpec((1, F_OUT), lambda i: (0, 0)),
        ],
        out_specs=pl.BlockSpec((BM, F_OUT), lambda i: (i, 0)),
        out_shape=jax.ShapeDtypeStruct((N, F_OUT), jnp.float32),
        scratch_shapes=[pltpu.VMEM((N, F_OUT), jnp.float32)],
        compiler_params=pltpu.CompilerParams(
            dimension_semantics=("arbitrary",),
        ),
    )(adj, input, W, b2)
    return output
